# baseline (speedup 1.0000x reference)
import functools

import jax
import jax.numpy as jnp
import numpy as np
from jax.experimental import pallas as pl
from jax.experimental.pallas import tpu as pltpu
from jax.experimental.pallas import tpu_sc as plsc

E = 64
K = 2
H = 768
INTER = 768
T = 64
F2 = 2 * INTER

_BIG = 1e30


@functools.cache
def _compress_matrix():
    s = np.zeros((F2, INTER), dtype=np.float32)
    s[2 * np.arange(INTER), np.arange(INTER)] = 1.0
    return jnp.asarray(s)


def _fiota(shape, dim):
    return jax.lax.broadcasted_iota(jnp.int32, shape, dim).astype(jnp.float32)


def _routing_body(x_ref, rw_ref, rb_ref, info_ref, elist_ref):
    x = x_ref[:, 0, :]
    rw = rw_ref[...]
    logits = jax.lax.dot_general(
        x, rw, (((1,), (1,)), ((), ())), preferred_element_type=jnp.float32)
    logits = logits + rb_ref[...]

    lane_f = _fiota((T, E), 1)

    m1 = jnp.max(logits, axis=1, keepdims=True)
    i1 = jnp.min(jnp.where(logits == m1, lane_f, _BIG), axis=1, keepdims=True)
    sel1 = lane_f == i1
    masked = jnp.where(sel1, -_BIG, logits)
    m2 = jnp.max(masked, axis=1, keepdims=True)
    i2 = jnp.min(jnp.where(masked == m2, lane_f, _BIG), axis=1, keepdims=True)
    sel2 = lane_f == i2

    e2 = jnp.exp(m2 - m1)
    w1 = 1.0 / (1.0 + e2)
    w2 = e2 / (1.0 + e2)
    info_ref[:, 0:1] = i1
    info_ref[:, 1:2] = i2
    info_ref[:, 2:3] = w1
    info_ref[:, 3:4] = w2
    info_ref[:, 4:16] = jnp.zeros((T, 12), jnp.float32)

    selected = jnp.logical_or(sel1, sel2).astype(jnp.float32)
    count = jnp.sum(selected, axis=0, keepdims=True)
    a = (count > 0.0).astype(jnp.float32)

    r = _fiota((E, E), 0)
    c = _fiota((E, E), 1)
    lower = (r <= c).astype(jnp.float32)
    cum = jax.lax.dot_general(
        a, lower, (((1,), (0,)), ((), ())), preferred_element_type=jnp.float32)
    nact = cum[:, E - 1:E]
    cum_i = jax.lax.dot_general(
        1.0 - a, lower, (((1,), (0,)), ((), ())),
        preferred_element_type=jnp.float32)
    pos = jnp.where(a > 0.0, cum - 1.0, nact + cum_i - 1.0)

    ident = (r == c).astype(jnp.float32)
    pos_col = jax.lax.dot_general(
        ident, pos, (((1,), (1,)), ((), ())), preferred_element_type=jnp.float32)
    a_col = jax.lax.dot_general(
        ident, a, (((1,), (1,)), ((), ())), preferred_element_type=jnp.float32)

    e_row = _fiota((1, E), 1)
    last_active = jnp.sum(
        jnp.where(jnp.logical_and(a > 0.0, pos == nact - 1.0), e_row, 0.0),
        axis=1, keepdims=True)

    e_sub = _fiota((E, E), 0)
    p_lane = _fiota((E, E), 1)
    ind = jnp.logical_and(pos_col == p_lane, a_col > 0.0)
    elist_active = jnp.sum(jnp.where(ind, e_sub, 0.0), axis=0, keepdims=True)
    p_row = _fiota((1, E), 1)
    elist = jnp.where(p_row < nact, elist_active, last_active)
    elist_ref[...] = elist.astype(jnp.int32)


def _sc_scatter_body(info_hbm, scores_hbm, info_v, sc_v):
    cid = jax.lax.axis_index("c")
    sid = jax.lax.axis_index("s")
    nv = E // 16

    @pl.when(jnp.logical_and(cid == 0, sid == 0))
    def _():
        pltpu.sync_copy(info_hbm, info_v)
        zero16 = jnp.zeros((16,), jnp.float32)
        iota_f = jax.lax.broadcasted_iota(jnp.int32, (16,), 0).astype(
            jnp.float32)

        def token(t, carry):
            row = info_v[t, pl.ds(0, 16)]
            i1 = row[0]
            i2 = row[1]
            w1 = row[2]
            w2 = row[3]
            for j in range(nv):
                idxs = iota_f + float(16 * j)
                z = jnp.where(idxs == i1, w1,
                              jnp.where(idxs == i2, w2, zero16))
                sc_v[t, pl.ds(16 * j, 16)] = z
            return carry

        jax.lax.fori_loop(0, T, token, 0)
        pltpu.sync_copy(sc_v, scores_hbm)


def _expert_body(el_ref, x_ref, info_ref, wgu_a_ref, wgu_b_ref, bgu_ref,
                 wd_ref, bd_ref, s_ref, out_ref):
    i = pl.program_id(0)
    e = el_ref[0, i]
    prev = el_ref[0, jnp.maximum(i - 1, 0)]
    valid = jnp.logical_or(i == 0, e != prev)

    @pl.when(valid)
    def _():
        x = x_ref[:, 0, :]
        gu = jax.lax.dot_general(
            x[:, 0:H // 2], wgu_a_ref[0], (((1,), (0,)), ((), ())),
            preferred_element_type=jnp.float32)
        gu = gu + jax.lax.dot_general(
            x[:, H // 2:H], wgu_b_ref[0], (((1,), (0,)), ((), ())),
            preferred_element_type=jnp.float32)
        gu = gu + bgu_ref[pl.ds(e, 1), :]
        lane = jax.lax.broadcasted_iota(jnp.int32, (T, F2), 1)
        even = (lane & 1) == 0
        up_sh = pltpu.roll(gu, F2 - 1, 1)
        gate = jnp.minimum(gu, 7.0)
        up = jnp.clip(up_sh, -7.0, 7.0)
        glu = gate * jax.nn.sigmoid(gate * 1.702)
        act2 = jnp.where(even, (up + 1.0) * glu, 0.0)
        act = jax.lax.dot_general(
            act2, s_ref[...], (((1,), (0,)), ((), ())),
            preferred_element_type=jnp.float32)
        oute = jax.lax.dot_general(
            act, wd_ref[0], (((1,), (0,)), ((), ())),
            preferred_element_type=jnp.float32)
        oute = oute + bd_ref[pl.ds(e, 1), :]
        e_f = e.astype(jnp.float32)
        sel1 = (info_ref[:, 0:1] == e_f).astype(jnp.float32)
        sel2 = (info_ref[:, 1:2] == e_f).astype(jnp.float32)
        col = info_ref[:, 2:3] * sel1 + info_ref[:, 3:4] * sel2
        contrib = (oute * col).reshape(T, 1, H)

        @pl.when(i == 0)
        def _():
            out_ref[...] = contrib

        @pl.when(i > 0)
        def _():
            out_ref[...] += contrib


@jax.jit
def kernel(hidden_states, router_weight, router_bias, gate_up_proj,
           gate_up_proj_bias, down_proj, down_proj_bias):
    b, s, h = hidden_states.shape

    info, elist = pl.pallas_call(
        _routing_body,
        out_shape=(
            jax.ShapeDtypeStruct((T, 16), jnp.float32),
            jax.ShapeDtypeStruct((1, E), jnp.int32),
        ),
    )(hidden_states, router_weight, router_bias.reshape(1, E))

    grid_spec = pltpu.PrefetchScalarGridSpec(
        num_scalar_prefetch=1,
        grid=(E,),
        in_specs=[
            pl.BlockSpec((T, 1, H), lambda i, el: (0, 0, 0)),
            pl.BlockSpec((T, 16), lambda i, el: (0, 0)),
            pl.BlockSpec((1, H // 2, F2), lambda i, el: (el[0, i], 0, 0)),
            pl.BlockSpec((1, H // 2, F2), lambda i, el: (el[0, i], 1, 0)),
            pl.BlockSpec((E, F2), lambda i, el: (0, 0)),
            pl.BlockSpec((1, INTER, H), lambda i, el: (el[0, i], 0, 0)),
            pl.BlockSpec((E, H), lambda i, el: (0, 0)),
            pl.BlockSpec((F2, INTER), lambda i, el: (0, 0)),
        ],
        out_specs=pl.BlockSpec((T, 1, H), lambda i, el: (0, 0, 0)),
    )
    out = pl.pallas_call(
        _expert_body,
        grid_spec=grid_spec,
        out_shape=jax.ShapeDtypeStruct((T, 1, H), jnp.float32),
        compiler_params=pltpu.CompilerParams(
            dimension_semantics=("arbitrary",)),
    )(elist, hidden_states, info, gate_up_proj, gate_up_proj,
      gate_up_proj_bias, down_proj,
      down_proj_bias, _compress_matrix())

    scores = pl.kernel(
        _sc_scatter_body,
        out_type=jax.ShapeDtypeStruct((T, E), jnp.float32),
        mesh=plsc.VectorSubcoreMesh(core_axis_name="c", subcore_axis_name="s"),
        scratch_types=[
            pltpu.VMEM((T, 16), jnp.float32),
            pltpu.VMEM((T, E), jnp.float32),
        ],
    )(info)

    return out, scores


# speedup vs baseline: 1.0237x; 1.0237x over previous
import functools

import jax
import jax.numpy as jnp
import numpy as np
from jax.experimental import pallas as pl
from jax.experimental.pallas import tpu as pltpu
from jax.experimental.pallas import tpu_sc as plsc

E = 64
K = 2
H = 768
INTER = 768
T = 64
F2 = 2 * INTER

_BIG = 1e30


@functools.cache
def _compress_matrix():
    s = np.zeros((F2, INTER), dtype=np.float32)
    s[2 * np.arange(INTER), np.arange(INTER)] = 1.0
    return jnp.asarray(s)


def _fiota(shape, dim):
    return jax.lax.broadcasted_iota(jnp.int32, shape, dim).astype(jnp.float32)


def _routing_body(x_ref, rw_ref, rb_ref, info_ref, elist_ref):
    x = x_ref[:, 0, :]
    rw = rw_ref[...]
    logits = jax.lax.dot_general(
        x, rw, (((1,), (1,)), ((), ())), preferred_element_type=jnp.float32)
    logits = logits + rb_ref[...]

    lane_f = _fiota((T, E), 1)

    m1 = jnp.max(logits, axis=1, keepdims=True)
    i1 = jnp.min(jnp.where(logits == m1, lane_f, _BIG), axis=1, keepdims=True)
    sel1 = lane_f == i1
    masked = jnp.where(sel1, -_BIG, logits)
    m2 = jnp.max(masked, axis=1, keepdims=True)
    i2 = jnp.min(jnp.where(masked == m2, lane_f, _BIG), axis=1, keepdims=True)
    sel2 = lane_f == i2

    e2 = jnp.exp(m2 - m1)
    w1 = 1.0 / (1.0 + e2)
    w2 = e2 / (1.0 + e2)
    info_ref[:, 0:1] = i1
    info_ref[:, 1:2] = i2
    info_ref[:, 2:3] = w1
    info_ref[:, 3:4] = w2
    info_ref[:, 4:16] = jnp.zeros((T, 12), jnp.float32)

    selected = jnp.logical_or(sel1, sel2).astype(jnp.float32)
    count = jnp.sum(selected, axis=0, keepdims=True)
    a = (count > 0.0).astype(jnp.float32)

    r = _fiota((E, E), 0)
    c = _fiota((E, E), 1)
    lower = (r <= c).astype(jnp.float32)
    cum = jax.lax.dot_general(
        a, lower, (((1,), (0,)), ((), ())), preferred_element_type=jnp.float32)
    nact = cum[:, E - 1:E]
    cum_i = jax.lax.dot_general(
        1.0 - a, lower, (((1,), (0,)), ((), ())),
        preferred_element_type=jnp.float32)
    pos = jnp.where(a > 0.0, cum - 1.0, nact + cum_i - 1.0)

    ident = (r == c).astype(jnp.float32)
    pos_col = jax.lax.dot_general(
        ident, pos, (((1,), (1,)), ((), ())), preferred_element_type=jnp.float32)
    a_col = jax.lax.dot_general(
        ident, a, (((1,), (1,)), ((), ())), preferred_element_type=jnp.float32)

    e_row = _fiota((1, E), 1)
    last_active = jnp.sum(
        jnp.where(jnp.logical_and(a > 0.0, pos == nact - 1.0), e_row, 0.0),
        axis=1, keepdims=True)

    e_sub = _fiota((E, E), 0)
    p_lane = _fiota((E, E), 1)
    ind = jnp.logical_and(pos_col == p_lane, a_col > 0.0)
    elist_active = jnp.sum(jnp.where(ind, e_sub, 0.0), axis=0, keepdims=True)
    p_row = _fiota((1, E), 1)
    elist = jnp.where(p_row < nact, elist_active, last_active)
    elist_ref[...] = elist.astype(jnp.int32)


def _sc_scatter_body(info_hbm, scores_hbm, info_v, sc_v):
    cid = jax.lax.axis_index("c")
    sid = jax.lax.axis_index("s")
    nv = E // 16

    @pl.when(jnp.logical_and(cid == 0, sid == 0))
    def _():
        pltpu.sync_copy(info_hbm, info_v)
        zero16 = jnp.zeros((16,), jnp.float32)
        iota_f = jax.lax.broadcasted_iota(jnp.int32, (16,), 0).astype(
            jnp.float32)

        def token(t, carry):
            row = info_v[t, pl.ds(0, 16)]
            i1 = row[0]
            i2 = row[1]
            w1 = row[2]
            w2 = row[3]
            for j in range(nv):
                idxs = iota_f + float(16 * j)
                z = jnp.where(idxs == i1, w1,
                              jnp.where(idxs == i2, w2, zero16))
                sc_v[t, pl.ds(16 * j, 16)] = z
            return carry

        jax.lax.fori_loop(0, T, token, 0)
        pltpu.sync_copy(sc_v, scores_hbm)


def _expert_body(el_ref, x_ref, info_ref, wgu_ref, bgu_ref, wd_ref, bd_ref,
                 s_ref, out_ref):
    i = pl.program_id(0)
    e = el_ref[0, i]
    prev = el_ref[0, jnp.maximum(i - 1, 0)]
    valid = jnp.logical_or(i == 0, e != prev)

    @pl.when(valid)
    def _():
        x = x_ref[:, 0, :]
        gu = jax.lax.dot_general(
            x, wgu_ref[0], (((1,), (0,)), ((), ())),
            preferred_element_type=jnp.float32)
        gu = gu + bgu_ref[pl.ds(e, 1), :]
        lane = jax.lax.broadcasted_iota(jnp.int32, (T, F2), 1)
        even = (lane & 1) == 0
        up_sh = pltpu.roll(gu, F2 - 1, 1)
        gate = jnp.minimum(gu, 7.0)
        up = jnp.clip(up_sh, -7.0, 7.0)
        glu = gate * jax.nn.sigmoid(gate * 1.702)
        act2 = jnp.where(even, (up + 1.0) * glu, 0.0)
        act = jax.lax.dot_general(
            act2, s_ref[...], (((1,), (0,)), ((), ())),
            preferred_element_type=jnp.float32)
        oute = jax.lax.dot_general(
            act, wd_ref[0], (((1,), (0,)), ((), ())),
            preferred_element_type=jnp.float32)
        oute = oute + bd_ref[pl.ds(e, 1), :]
        e_f = e.astype(jnp.float32)
        sel1 = (info_ref[:, 0:1] == e_f).astype(jnp.float32)
        sel2 = (info_ref[:, 1:2] == e_f).astype(jnp.float32)
        col = info_ref[:, 2:3] * sel1 + info_ref[:, 3:4] * sel2
        contrib = (oute * col).reshape(T, 1, H)

        @pl.when(i == 0)
        def _():
            out_ref[...] = contrib

        @pl.when(i > 0)
        def _():
            out_ref[...] += contrib


@jax.jit
def kernel(hidden_states, router_weight, router_bias, gate_up_proj,
           gate_up_proj_bias, down_proj, down_proj_bias):
    b, s, h = hidden_states.shape

    info, elist = pl.pallas_call(
        _routing_body,
        out_shape=(
            jax.ShapeDtypeStruct((T, 16), jnp.float32),
            jax.ShapeDtypeStruct((1, E), jnp.int32),
        ),
    )(hidden_states, router_weight, router_bias.reshape(1, E))

    grid_spec = pltpu.PrefetchScalarGridSpec(
        num_scalar_prefetch=1,
        grid=(E,),
        in_specs=[
            pl.BlockSpec((T, 1, H), lambda i, el: (0, 0, 0)),
            pl.BlockSpec((T, 16), lambda i, el: (0, 0)),
            pl.BlockSpec((1, H, F2), lambda i, el: (el[0, i], 0, 0)),
            pl.BlockSpec((E, F2), lambda i, el: (0, 0)),
            pl.BlockSpec((1, INTER, H), lambda i, el: (el[0, i], 0, 0)),
            pl.BlockSpec((E, H), lambda i, el: (0, 0)),
            pl.BlockSpec((F2, INTER), lambda i, el: (0, 0)),
        ],
        out_specs=pl.BlockSpec((T, 1, H), lambda i, el: (0, 0, 0)),
    )
    out = pl.pallas_call(
        _expert_body,
        grid_spec=grid_spec,
        out_shape=jax.ShapeDtypeStruct((T, 1, H), jnp.float32),
        compiler_params=pltpu.CompilerParams(
            dimension_semantics=("arbitrary",)),
    )(elist, hidden_states, info, gate_up_proj,
      gate_up_proj_bias, down_proj,
      down_proj_bias, _compress_matrix())

    scores = pl.kernel(
        _sc_scatter_body,
        out_type=jax.ShapeDtypeStruct((T, E), jnp.float32),
        mesh=plsc.VectorSubcoreMesh(core_axis_name="c", subcore_axis_name="s"),
        scratch_types=[
            pltpu.VMEM((T, 16), jnp.float32),
            pltpu.VMEM((T, E), jnp.float32),
        ],
    )(info)

    return out, scores
